# Initial kernel scaffold; baseline (speedup 1.0000x reference)
import functools
import jax
import jax.numpy as jnp
from jax import lax
from jax.experimental import pallas as pl
from jax.experimental.pallas import tpu as pltpu
from jax.experimental.pallas import tpu_sc as plsc

N = 10000
E = 320000
D = 128
DE = 16
G = 64
H = 32
MSG = 4
OUT = 8

MP = 16
NC = 2
NS = 16
NW = NC * NS
CH = 128
KCH = 80
EPT = KCH * CH
EP = NW * EPT
NSH = N + MP
RPT = NSH // NS

_mesh = plsc.VectorSubcoreMesh(core_axis_name="c", subcore_axis_name="s")


def _silu(v):
    return v * jax.nn.sigmoid(v)


@functools.partial(
    pl.kernel,
    out_type=(
        jax.ShapeDtypeStruct((EP, H), jnp.float32),
        jax.ShapeDtypeStruct((EP, H), jnp.float32),
    ),
    mesh=_mesh,
    scratch_types=[
        pltpu.VMEM((KCH, CH), jnp.int32),
        pltpu.VMEM((KCH, CH), jnp.int32),
        pltpu.VMEM((CH, H), jnp.float32),
        pltpu.VMEM((CH, H), jnp.float32),
        pltpu.SemaphoreType.DMA,
    ],
)
def _sc_gather(a_hbm, b_hbm, i0_hbm, i1_hbm, g0_hbm, g1_hbm,
               i0_v, i1_v, r0_v, r1_v, sem):
    c = lax.axis_index("c")
    s = lax.axis_index("s")
    wid = c * NS + s
    pltpu.sync_copy(i0_hbm.at[wid], i0_v)
    pltpu.sync_copy(i1_hbm.at[wid], i1_v)

    def chunk(j, carry):
        base = wid * EPT + j * CH
        pltpu.async_copy(a_hbm.at[i0_v.at[j]], r0_v, sem).wait()
        pltpu.sync_copy(r0_v, g0_hbm.at[pl.ds(base, CH)])
        pltpu.async_copy(b_hbm.at[i1_v.at[j]], r1_v, sem).wait()
        pltpu.sync_copy(r1_v, g1_hbm.at[pl.ds(base, CH)])
        return carry

    lax.fori_loop(0, KCH, chunk, 0)


@functools.partial(
    pl.kernel,
    out_type=jax.ShapeDtypeStruct((NC, NSH, MP), jnp.float32),
    mesh=_mesh,
    scratch_types=[
        pltpu.VMEM((KCH, CH), jnp.int32),
        pltpu.VMEM((CH, MP), jnp.float32),
        pltpu.VMEM((RPT, MP), jnp.float32),
        pltpu.VMEM_SHARED((NSH, MP), jnp.float32),
    ],
)
def _sc_scatter(m_hbm, i0_hbm, p_hbm, i0_v, r_v, st_v, acc_sh):
    c = lax.axis_index("c")
    s = lax.axis_index("s")
    wid = c * NS + s

    def zrow(i, carry):
        st_v[i, :] = jnp.zeros((MP,), jnp.float32)
        return carry

    lax.fori_loop(0, RPT, zrow, 0)
    pltpu.sync_copy(st_v, acc_sh.at[pl.ds(s * RPT, RPT)])
    plsc.subcore_barrier()

    pltpu.sync_copy(i0_hbm.at[wid], i0_v)

    def chunk(j, carry):
        base = wid * EPT + j * CH
        pltpu.sync_copy(m_hbm.at[pl.ds(base, CH)], r_v)
        pltpu.sync_copy(r_v, acc_sh.at[i0_v.at[j]], add=True)
        return carry

    lax.fori_loop(0, KCH, chunk, 0)
    plsc.subcore_barrier()

    pltpu.sync_copy(acc_sh.at[pl.ds(s * RPT, RPT)], st_v)
    pltpu.sync_copy(st_v, p_hbm.at[c, pl.ds(s * RPT, RPT)])


_BN = 2000
_BE = 2048


def _proj_body(x_ref, wa_ref, wb_ref, a_ref, b_ref):
    xb = x_ref[...]
    a_ref[...] = jnp.dot(xb, wa_ref[...], preferred_element_type=jnp.float32)
    b_ref[...] = jnp.dot(xb, wb_ref[...], preferred_element_type=jnp.float32)


def _proj(h, wa, wb):
    return pl.pallas_call(
        _proj_body,
        grid=(N // _BN,),
        in_specs=[
            pl.BlockSpec((_BN, D), lambda i: (i, 0)),
            pl.BlockSpec((D, H), lambda i: (0, 0)),
            pl.BlockSpec((D, H), lambda i: (0, 0)),
        ],
        out_specs=[
            pl.BlockSpec((_BN, H), lambda i: (i, 0)),
            pl.BlockSpec((_BN, H), lambda i: (i, 0)),
        ],
        out_shape=[
            jax.ShapeDtypeStruct((N, H), jnp.float32),
            jax.ShapeDtypeStruct((N, H), jnp.float32),
        ],
    )(h, wa, wb)


def _edge_body(g0_ref, g1_ref, ea_ref, w1c_ref, b1_ref, w2_ref, b2_ref, m_ref):
    sv = (g0_ref[...] + g1_ref[...]
          + jnp.dot(ea_ref[...], w1c_ref[...], preferred_element_type=jnp.float32)
          + b1_ref[...])
    t = _silu(sv)
    m_ref[...] = _silu(jnp.dot(t, w2_ref[...], preferred_element_type=jnp.float32)
                       + b2_ref[...])


def _edge(g0, g1, eap, w1c, b1, w2p, b2p):
    return pl.pallas_call(
        _edge_body,
        grid=(EP // _BE,),
        in_specs=[
            pl.BlockSpec((_BE, H), lambda i: (i, 0)),
            pl.BlockSpec((_BE, H), lambda i: (i, 0)),
            pl.BlockSpec((_BE, DE), lambda i: (i, 0)),
            pl.BlockSpec((DE, H), lambda i: (0, 0)),
            pl.BlockSpec((1, H), lambda i: (0, 0)),
            pl.BlockSpec((H, MP), lambda i: (0, 0)),
            pl.BlockSpec((1, MP), lambda i: (0, 0)),
        ],
        out_specs=pl.BlockSpec((_BE, MP), lambda i: (i, 0)),
        out_shape=jax.ShapeDtypeStruct((EP, MP), jnp.float32),
    )(g0, g1, eap, w1c, b1, w2p, b2p)


def _node_proj_body(h_ref, p0_ref, p1_ref, v1a_ref, v1b_ref, b1_ref,
                    w2_ref, b2_ref, wa_ref, wb_ref, h_out, a_out, b_out):
    hb = h_ref[...]
    sums = p0_ref[0] + p1_ref[0]
    t = _silu(jnp.dot(hb, v1a_ref[...], preferred_element_type=jnp.float32)
              + jnp.dot(sums, v1b_ref[...], preferred_element_type=jnp.float32)
              + b1_ref[...])
    hn = _silu(_silu(jnp.dot(t, w2_ref[...], preferred_element_type=jnp.float32)
                     + b2_ref[...]))
    h_out[...] = hn
    a_out[...] = jnp.dot(hn, wa_ref[...], preferred_element_type=jnp.float32)
    b_out[...] = jnp.dot(hn, wb_ref[...], preferred_element_type=jnp.float32)


def _node_final_body(h_ref, p0_ref, p1_ref, v1a_ref, v1b_ref, b1_ref,
                     w2_ref, b2_ref, h_out):
    hb = h_ref[...]
    sums = p0_ref[0] + p1_ref[0]
    t = _silu(jnp.dot(hb, v1a_ref[...], preferred_element_type=jnp.float32)
              + jnp.dot(sums, v1b_ref[...], preferred_element_type=jnp.float32)
              + b1_ref[...])
    h_out[...] = _silu(_silu(jnp.dot(t, w2_ref[...], preferred_element_type=jnp.float32)
                             + b2_ref[...]))


def _node_specs():
    return [
        pl.BlockSpec((_BN, D), lambda i: (i, 0)),
        pl.BlockSpec((1, _BN, MP), lambda i: (0, i, 0)),
        pl.BlockSpec((1, _BN, MP), lambda i: (1, i, 0)),
        pl.BlockSpec((D, H), lambda i: (0, 0)),
        pl.BlockSpec((MP, H), lambda i: (0, 0)),
        pl.BlockSpec((1, H), lambda i: (0, 0)),
        pl.BlockSpec((H, D), lambda i: (0, 0)),
        pl.BlockSpec((1, D), lambda i: (0, 0)),
    ]


def _node_proj(h, p, v1a, v1b, b1, w2, b2, wa, wb):
    return pl.pallas_call(
        _node_proj_body,
        grid=(N // _BN,),
        in_specs=_node_specs() + [
            pl.BlockSpec((D, H), lambda i: (0, 0)),
            pl.BlockSpec((D, H), lambda i: (0, 0)),
        ],
        out_specs=[
            pl.BlockSpec((_BN, D), lambda i: (i, 0)),
            pl.BlockSpec((_BN, H), lambda i: (i, 0)),
            pl.BlockSpec((_BN, H), lambda i: (i, 0)),
        ],
        out_shape=[
            jax.ShapeDtypeStruct((N, D), jnp.float32),
            jax.ShapeDtypeStruct((N, H), jnp.float32),
            jax.ShapeDtypeStruct((N, H), jnp.float32),
        ],
    )(h, p, p, v1a, v1b, b1, w2, b2, wa, wb)


def _node_final(h, p, v1a, v1b, b1, w2, b2):
    return pl.pallas_call(
        _node_final_body,
        grid=(N // _BN,),
        in_specs=_node_specs(),
        out_specs=pl.BlockSpec((_BN, D), lambda i: (i, 0)),
        out_shape=jax.ShapeDtypeStruct((N, D), jnp.float32),
    )(h, p, p, v1a, v1b, b1, w2, b2)


def _final_body(ui_ref, ue_ref, bt_ref, w1a_ref, w1b_ref, b1_ref,
                w2_ref, b2_ref, o_ref, acc_i, acc_e, cnt):
    k = pl.program_id(0)

    @pl.when(k == 0)
    def _():
        acc_i[...] = jnp.zeros_like(acc_i)
        acc_e[...] = jnp.zeros_like(acc_e)
        cnt[...] = jnp.zeros_like(cnt)

    bt = bt_ref[...]
    seg = lax.broadcasted_iota(jnp.float32, (G, _BN), 0)
    oh = jnp.where(seg == bt, 1.0, 0.0)
    acc_i[...] += jnp.dot(oh, ui_ref[...], preferred_element_type=jnp.float32)
    acc_e[...] += jnp.dot(oh, ue_ref[...], preferred_element_type=jnp.float32)
    cnt[...] += jnp.broadcast_to(jnp.sum(oh, axis=1, keepdims=True), (G, D))

    @pl.when(k == (N // _BN) - 1)
    def _():
        c = jnp.maximum(cnt[...], 1.0)
        mi = acc_i[...] / c
        me = acc_e[...] / c
        t = _silu(jnp.dot(mi, w1a_ref[...], preferred_element_type=jnp.float32)
                  + jnp.dot(me, w1b_ref[...], preferred_element_type=jnp.float32)
                  + b1_ref[...])
        o_ref[...] = _silu(jnp.dot(t, w2_ref[...], preferred_element_type=jnp.float32)
                           + b2_ref[...])


def _final(ui, ue, btf, w1a, w1b, b1, w2, b2):
    return pl.pallas_call(
        _final_body,
        grid=(N // _BN,),
        in_specs=[
            pl.BlockSpec((_BN, D), lambda i: (i, 0)),
            pl.BlockSpec((_BN, D), lambda i: (i, 0)),
            pl.BlockSpec((1, _BN), lambda i: (0, i)),
            pl.BlockSpec((D, H), lambda i: (0, 0)),
            pl.BlockSpec((D, H), lambda i: (0, 0)),
            pl.BlockSpec((1, H), lambda i: (0, 0)),
            pl.BlockSpec((H, OUT), lambda i: (0, 0)),
            pl.BlockSpec((1, OUT), lambda i: (0, 0)),
        ],
        out_specs=pl.BlockSpec((G, OUT), lambda i: (0, 0)),
        out_shape=jax.ShapeDtypeStruct((G, OUT), jnp.float32),
        scratch_shapes=[
            pltpu.VMEM((G, D), jnp.float32),
            pltpu.VMEM((G, D), jnp.float32),
            pltpu.VMEM((G, D), jnp.float32),
        ],
    )(ui, ue, btf, w1a, w1b, b1, w2, b2)


def _prep_conv(p):
    w1 = p["edge"]["W1"]
    wa = w1[:D]
    wb = w1[D:2 * D]
    w1c = w1[2 * D:]
    b1 = p["edge"]["b1"].reshape(1, H)
    w2p = jnp.zeros((H, MP), jnp.float32).at[:, :MSG].set(p["edge"]["W2"])
    b2p = jnp.zeros((1, MP), jnp.float32).at[0, :MSG].set(p["edge"]["b2"])
    nw1 = p["node"]["W1"]
    v1a = nw1[:D]
    v1b = jnp.zeros((MP, H), jnp.float32).at[:MSG].set(nw1[D:])
    nb1 = p["node"]["b1"].reshape(1, H)
    nw2 = p["node"]["W2"]
    nb2 = p["node"]["b2"].reshape(1, D)
    return wa, wb, w1c, b1, w2p, b2p, v1a, v1b, nb1, nw2, nb2


def _branch(x, eidx, eattr, layers):
    pad = EP - E
    n0 = eidx[0].astype(jnp.int32)
    n1 = eidx[1].astype(jnp.int32)
    n0g = jnp.concatenate([n0, jnp.zeros((pad,), jnp.int32)]).reshape(NW, KCH, CH)
    n1g = jnp.concatenate([n1, jnp.zeros((pad,), jnp.int32)]).reshape(NW, KCH, CH)
    n0s = jnp.concatenate([n0, jnp.full((pad,), N, jnp.int32)]).reshape(NW, KCH, CH)
    eap = jnp.concatenate([eattr, jnp.zeros((pad, DE), jnp.float32)])

    prepped = [_prep_conv(p) for p in layers]
    h = x
    a, b = _proj(h, prepped[0][0], prepped[0][1])
    for li, pr in enumerate(prepped):
        wa, wb, w1c, b1, w2p, b2p, v1a, v1b, nb1, nw2, nb2 = pr
        g0, g1 = _sc_gather(a, b, n0g, n1g)
        m = _edge(g0, g1, eap, w1c, b1, w2p, b2p)
        psum = _sc_scatter(m, n0s)
        if li + 1 < len(prepped):
            h, a, b = _node_proj(h, psum, v1a, v1b, nb1, nw2, nb2,
                                 prepped[li + 1][0], prepped[li + 1][1])
        else:
            h = _node_final(h, psum, v1a, v1b, nb1, nw2, nb2)
    return h


def kernel(x, internal_edge_index, internal_edge_attr, edge_index, edge_attr,
           batch, internal_params, external_params, graph_params):
    upd_int = _branch(x, internal_edge_index, internal_edge_attr, internal_params)
    upd_ext = _branch(x, edge_index, edge_attr, external_params)
    btf = batch.astype(jnp.float32).reshape(1, N)
    gw1 = graph_params["W1"]
    out = _final(upd_int, upd_ext, btf,
                 gw1[:D], gw1[D:], graph_params["b1"].reshape(1, H),
                 graph_params["W2"], graph_params["b2"].reshape(1, OUT))
    return out


# speedup vs baseline: 2.1655x; 2.1655x over previous
import functools
import jax
import jax.numpy as jnp
from jax import lax
from jax.experimental import pallas as pl
from jax.experimental.pallas import tpu as pltpu
from jax.experimental.pallas import tpu_sc as plsc

N = 10000
E = 320000
D = 128
DE = 16
G = 64
H = 32
MSG = 4
OUT = 8

MP = 16
NC = 2
NS = 16
NW = NC * NS
CH = 128
KCH = 80
EPT = KCH * CH
EP = NW * EPT
NSH = N + MP
RPT = NSH // NS

def _silu(v):
    return v * jax.nn.sigmoid(v)


@functools.cache
def _sc_gather_kernel():
    mesh = plsc.VectorSubcoreMesh(core_axis_name="c", subcore_axis_name="s",
                                  num_cores=NC, num_subcores=NS)

    @functools.partial(
        pl.kernel,
        out_type=(
            jax.ShapeDtypeStruct((EP, H), jnp.float32),
            jax.ShapeDtypeStruct((EP, H), jnp.float32),
        ),
        mesh=mesh,
        compiler_params=pltpu.CompilerParams(use_tc_tiling_on_sc=False),
        scratch_types=[
            pltpu.VMEM((KCH, CH), jnp.int32),
            pltpu.VMEM((KCH, CH), jnp.int32),
            pltpu.VMEM((CH, H), jnp.float32),
            pltpu.VMEM((CH, H), jnp.float32),
            pltpu.SemaphoreType.DMA,
        ],
    )
    def gather(a_hbm, b_hbm, i0_hbm, i1_hbm, g0_hbm, g1_hbm,
               i0_v, i1_v, r0_v, r1_v, sem):
        c = lax.axis_index("c")
        s = lax.axis_index("s")
        wid = c * NS + s
        pltpu.sync_copy(i0_hbm.at[wid], i0_v)
        pltpu.sync_copy(i1_hbm.at[wid], i1_v)

        def chunk(j, carry):
            base = wid * EPT + j * CH
            pltpu.async_copy(a_hbm.at[i0_v.at[j]], r0_v, sem).wait()
            pltpu.sync_copy(r0_v, g0_hbm.at[pl.ds(base, CH)])
            pltpu.async_copy(b_hbm.at[i1_v.at[j]], r1_v, sem).wait()
            pltpu.sync_copy(r1_v, g1_hbm.at[pl.ds(base, CH)])
            return carry

        lax.fori_loop(0, KCH, chunk, 0)

    return gather


@functools.cache
def _sc_scatter_kernel():
    mesh = plsc.VectorSubcoreMesh(core_axis_name="c", subcore_axis_name="s",
                                  num_cores=NC, num_subcores=NS)

    @functools.partial(
        pl.kernel,
        out_type=jax.ShapeDtypeStruct((NC, NSH, MP), jnp.float32),
        mesh=mesh,
        compiler_params=pltpu.CompilerParams(use_tc_tiling_on_sc=False),
        scratch_types=[
            pltpu.VMEM((KCH, CH), jnp.int32),
            pltpu.VMEM((CH, MP), jnp.float32),
            pltpu.VMEM((RPT, MP), jnp.float32),
            pltpu.VMEM_SHARED((NSH, MP), jnp.float32),
        ],
    )
    def scatter(m_hbm, i0_hbm, p_hbm, i0_v, r_v, st_v, acc_sh):
        c = lax.axis_index("c")
        s = lax.axis_index("s")
        wid = c * NS + s

        def zrow(i, carry):
            st_v[i, :] = jnp.zeros((MP,), jnp.float32)
            return carry

        lax.fori_loop(0, RPT, zrow, 0)
        pltpu.sync_copy(st_v, acc_sh.at[pl.ds(s * RPT, RPT)])
        plsc.subcore_barrier()

        pltpu.sync_copy(i0_hbm.at[wid], i0_v)

        def chunk(j, carry):
            base = wid * EPT + j * CH
            pltpu.sync_copy(m_hbm.at[pl.ds(base, CH)], r_v)
            pltpu.sync_copy(r_v, acc_sh.at[i0_v.at[j]], add=True)
            return carry

        lax.fori_loop(0, KCH, chunk, 0)
        plsc.subcore_barrier()

        pltpu.sync_copy(acc_sh.at[pl.ds(s * RPT, RPT)], st_v)
        pltpu.sync_copy(st_v, p_hbm.at[c, pl.ds(s * RPT, RPT)])

    return scatter


def _sc_gather(a, b, i0, i1):
    return _sc_gather_kernel()(a, b, i0, i1)


def _sc_scatter(m, i0s):
    return _sc_scatter_kernel()(m, i0s)


_BN = 2000
_BE = 2048


def _proj_body(x_ref, wa_ref, wb_ref, a_ref, b_ref):
    xb = x_ref[...]
    a_ref[...] = jnp.dot(xb, wa_ref[...], preferred_element_type=jnp.float32)
    b_ref[...] = jnp.dot(xb, wb_ref[...], preferred_element_type=jnp.float32)


def _proj(h, wa, wb):
    return pl.pallas_call(
        _proj_body,
        grid=(N // _BN,),
        in_specs=[
            pl.BlockSpec((_BN, D), lambda i: (i, 0)),
            pl.BlockSpec((D, H), lambda i: (0, 0)),
            pl.BlockSpec((D, H), lambda i: (0, 0)),
        ],
        out_specs=[
            pl.BlockSpec((_BN, H), lambda i: (i, 0)),
            pl.BlockSpec((_BN, H), lambda i: (i, 0)),
        ],
        out_shape=[
            jax.ShapeDtypeStruct((N, H), jnp.float32),
            jax.ShapeDtypeStruct((N, H), jnp.float32),
        ],
    )(h, wa, wb)


def _edge_body(g0_ref, g1_ref, ea_ref, w1c_ref, b1_ref, w2_ref, b2_ref, m_ref):
    sv = (g0_ref[...] + g1_ref[...]
          + jnp.dot(ea_ref[...], w1c_ref[...], preferred_element_type=jnp.float32)
          + b1_ref[...])
    t = _silu(sv)
    m_ref[...] = _silu(jnp.dot(t, w2_ref[...], preferred_element_type=jnp.float32)
                       + b2_ref[...])


def _edge(g0, g1, eap, w1c, b1, w2p, b2p):
    return pl.pallas_call(
        _edge_body,
        grid=(EP // _BE,),
        in_specs=[
            pl.BlockSpec((_BE, H), lambda i: (i, 0)),
            pl.BlockSpec((_BE, H), lambda i: (i, 0)),
            pl.BlockSpec((_BE, DE), lambda i: (i, 0)),
            pl.BlockSpec((DE, H), lambda i: (0, 0)),
            pl.BlockSpec((1, H), lambda i: (0, 0)),
            pl.BlockSpec((H, MP), lambda i: (0, 0)),
            pl.BlockSpec((1, MP), lambda i: (0, 0)),
        ],
        out_specs=pl.BlockSpec((_BE, MP), lambda i: (i, 0)),
        out_shape=jax.ShapeDtypeStruct((EP, MP), jnp.float32),
    )(g0, g1, eap, w1c, b1, w2p, b2p)


def _node_proj_body(h_ref, p0_ref, p1_ref, v1a_ref, v1b_ref, b1_ref,
                    w2_ref, b2_ref, wa_ref, wb_ref, h_out, a_out, b_out):
    hb = h_ref[...]
    sums = p0_ref[0] + p1_ref[0]
    t = _silu(jnp.dot(hb, v1a_ref[...], preferred_element_type=jnp.float32)
              + jnp.dot(sums, v1b_ref[...], preferred_element_type=jnp.float32)
              + b1_ref[...])
    hn = _silu(_silu(jnp.dot(t, w2_ref[...], preferred_element_type=jnp.float32)
                     + b2_ref[...]))
    h_out[...] = hn
    a_out[...] = jnp.dot(hn, wa_ref[...], preferred_element_type=jnp.float32)
    b_out[...] = jnp.dot(hn, wb_ref[...], preferred_element_type=jnp.float32)


def _node_final_body(h_ref, p0_ref, p1_ref, v1a_ref, v1b_ref, b1_ref,
                     w2_ref, b2_ref, h_out):
    hb = h_ref[...]
    sums = p0_ref[0] + p1_ref[0]
    t = _silu(jnp.dot(hb, v1a_ref[...], preferred_element_type=jnp.float32)
              + jnp.dot(sums, v1b_ref[...], preferred_element_type=jnp.float32)
              + b1_ref[...])
    h_out[...] = _silu(_silu(jnp.dot(t, w2_ref[...], preferred_element_type=jnp.float32)
                             + b2_ref[...]))


def _node_specs():
    return [
        pl.BlockSpec((_BN, D), lambda i: (i, 0)),
        pl.BlockSpec((1, _BN, MP), lambda i: (0, i, 0)),
        pl.BlockSpec((1, _BN, MP), lambda i: (1, i, 0)),
        pl.BlockSpec((D, H), lambda i: (0, 0)),
        pl.BlockSpec((MP, H), lambda i: (0, 0)),
        pl.BlockSpec((1, H), lambda i: (0, 0)),
        pl.BlockSpec((H, D), lambda i: (0, 0)),
        pl.BlockSpec((1, D), lambda i: (0, 0)),
    ]


def _node_proj(h, p, v1a, v1b, b1, w2, b2, wa, wb):
    return pl.pallas_call(
        _node_proj_body,
        grid=(N // _BN,),
        in_specs=_node_specs() + [
            pl.BlockSpec((D, H), lambda i: (0, 0)),
            pl.BlockSpec((D, H), lambda i: (0, 0)),
        ],
        out_specs=[
            pl.BlockSpec((_BN, D), lambda i: (i, 0)),
            pl.BlockSpec((_BN, H), lambda i: (i, 0)),
            pl.BlockSpec((_BN, H), lambda i: (i, 0)),
        ],
        out_shape=[
            jax.ShapeDtypeStruct((N, D), jnp.float32),
            jax.ShapeDtypeStruct((N, H), jnp.float32),
            jax.ShapeDtypeStruct((N, H), jnp.float32),
        ],
    )(h, p, p, v1a, v1b, b1, w2, b2, wa, wb)


def _node_final(h, p, v1a, v1b, b1, w2, b2):
    return pl.pallas_call(
        _node_final_body,
        grid=(N // _BN,),
        in_specs=_node_specs(),
        out_specs=pl.BlockSpec((_BN, D), lambda i: (i, 0)),
        out_shape=jax.ShapeDtypeStruct((N, D), jnp.float32),
    )(h, p, p, v1a, v1b, b1, w2, b2)


def _final_body(ui_ref, ue_ref, bt_ref, w1a_ref, w1b_ref, b1_ref,
                w2_ref, b2_ref, o_ref, acc_i, acc_e, cnt):
    k = pl.program_id(0)

    @pl.when(k == 0)
    def _():
        acc_i[...] = jnp.zeros_like(acc_i)
        acc_e[...] = jnp.zeros_like(acc_e)
        cnt[...] = jnp.zeros_like(cnt)

    bt = bt_ref[:, :G]
    seg = lax.broadcasted_iota(jnp.int32, (_BN, G), 1)
    oh = jnp.where(seg == bt, 1.0, 0.0)
    dn = (((0,), (0,)), ((), ()))
    acc_i[...] += lax.dot_general(oh, ui_ref[...], dn,
                                  preferred_element_type=jnp.float32)
    acc_e[...] += lax.dot_general(oh, ue_ref[...], dn,
                                  preferred_element_type=jnp.float32)
    cnt[...] += lax.dot_general(oh, jnp.ones((_BN, D), jnp.float32), dn,
                                preferred_element_type=jnp.float32)

    @pl.when(k == (N // _BN) - 1)
    def _():
        c = jnp.maximum(cnt[...], 1.0)
        mi = acc_i[...] / c
        me = acc_e[...] / c
        t = _silu(jnp.dot(mi, w1a_ref[...], preferred_element_type=jnp.float32)
                  + jnp.dot(me, w1b_ref[...], preferred_element_type=jnp.float32)
                  + b1_ref[...])
        o_ref[...] = _silu(jnp.dot(t, w2_ref[...], preferred_element_type=jnp.float32)
                           + b2_ref[...])


def _final(ui, ue, btf, w1a, w1b, b1, w2, b2):
    return pl.pallas_call(
        _final_body,
        grid=(N // _BN,),
        in_specs=[
            pl.BlockSpec((_BN, D), lambda i: (i, 0)),
            pl.BlockSpec((_BN, D), lambda i: (i, 0)),
            pl.BlockSpec((_BN, D), lambda i: (i, 0)),
            pl.BlockSpec((D, H), lambda i: (0, 0)),
            pl.BlockSpec((D, H), lambda i: (0, 0)),
            pl.BlockSpec((1, H), lambda i: (0, 0)),
            pl.BlockSpec((H, OUT), lambda i: (0, 0)),
            pl.BlockSpec((1, OUT), lambda i: (0, 0)),
        ],
        out_specs=pl.BlockSpec((G, OUT), lambda i: (0, 0)),
        out_shape=jax.ShapeDtypeStruct((G, OUT), jnp.float32),
        scratch_shapes=[
            pltpu.VMEM((G, D), jnp.float32),
            pltpu.VMEM((G, D), jnp.float32),
            pltpu.VMEM((G, D), jnp.float32),
        ],
    )(ui, ue, btf, w1a, w1b, b1, w2, b2)


def _prep_conv(p):
    w1 = p["edge"]["W1"]
    wa = w1[:D]
    wb = w1[D:2 * D]
    w1c = w1[2 * D:]
    b1 = p["edge"]["b1"].reshape(1, H)
    w2p = jnp.zeros((H, MP), jnp.float32).at[:, :MSG].set(p["edge"]["W2"])
    b2p = jnp.zeros((1, MP), jnp.float32).at[0, :MSG].set(p["edge"]["b2"])
    nw1 = p["node"]["W1"]
    v1a = nw1[:D]
    v1b = jnp.zeros((MP, H), jnp.float32).at[:MSG].set(nw1[D:])
    nb1 = p["node"]["b1"].reshape(1, H)
    nw2 = p["node"]["W2"]
    nb2 = p["node"]["b2"].reshape(1, D)
    return wa, wb, w1c, b1, w2p, b2p, v1a, v1b, nb1, nw2, nb2


def _branch(x, eidx, eattr, layers):
    pad = EP - E
    n0 = eidx[0].astype(jnp.int32)
    n1 = eidx[1].astype(jnp.int32)
    n0g = jnp.concatenate([n0, jnp.zeros((pad,), jnp.int32)]).reshape(NW, KCH, CH)
    n1g = jnp.concatenate([n1, jnp.zeros((pad,), jnp.int32)]).reshape(NW, KCH, CH)
    n0s = jnp.concatenate([n0, jnp.full((pad,), N, jnp.int32)]).reshape(NW, KCH, CH)
    eap = jnp.concatenate([eattr, jnp.zeros((pad, DE), jnp.float32)])

    prepped = [_prep_conv(p) for p in layers]
    h = x
    a, b = _proj(h, prepped[0][0], prepped[0][1])
    for li, pr in enumerate(prepped):
        wa, wb, w1c, b1, w2p, b2p, v1a, v1b, nb1, nw2, nb2 = pr
        g0, g1 = _sc_gather(a, b, n0g, n1g)
        m = _edge(g0, g1, eap, w1c, b1, w2p, b2p)
        psum = _sc_scatter(m, n0s)
        if li + 1 < len(prepped):
            h, a, b = _node_proj(h, psum, v1a, v1b, nb1, nw2, nb2,
                                 prepped[li + 1][0], prepped[li + 1][1])
        else:
            h = _node_final(h, psum, v1a, v1b, nb1, nw2, nb2)
    return h


def kernel(x, internal_edge_index, internal_edge_attr, edge_index, edge_attr,
           batch, internal_params, external_params, graph_params):
    upd_int = _branch(x, internal_edge_index, internal_edge_attr, internal_params)
    upd_ext = _branch(x, edge_index, edge_attr, external_params)
    btf = jnp.broadcast_to(batch.astype(jnp.int32)[:, None], (N, D))
    gw1 = graph_params["W1"]
    out = _final(upd_int, upd_ext, btf,
                 gw1[:D], gw1[D:], graph_params["b1"].reshape(1, H),
                 graph_params["W2"], graph_params["b2"].reshape(1, OUT))
    return out


# speedup vs baseline: 2.5468x; 1.1761x over previous
import functools
import jax
import jax.numpy as jnp
from jax import lax
from jax.experimental import pallas as pl
from jax.experimental.pallas import tpu as pltpu
from jax.experimental.pallas import tpu_sc as plsc

N = 10000
E = 320000
D = 128
DE = 16
G = 64
H = 32
MSG = 4
OUT = 8

MP = 16
NC = 2
NS = 16
NW = NC * NS
CH = 128
KCH = 80
EPT = KCH * CH
EP = NW * EPT
NSH = N + MP
RPT = NSH // NS

def _silu(v):
    return v * jax.nn.sigmoid(v)


GK = 4
GR = GK * CH
NG = KCH // GK


@functools.cache
def _sc_gather_kernel():
    mesh = plsc.VectorSubcoreMesh(core_axis_name="c", subcore_axis_name="s",
                                  num_cores=NC, num_subcores=NS)

    @functools.partial(
        pl.kernel,
        out_type=(
            jax.ShapeDtypeStruct((EP, H), jnp.float32),
            jax.ShapeDtypeStruct((EP, H), jnp.float32),
        ),
        mesh=mesh,
        compiler_params=pltpu.CompilerParams(use_tc_tiling_on_sc=False),
        scratch_types=[
            pltpu.VMEM((EPT,), jnp.int32),
            pltpu.VMEM((EPT,), jnp.int32),
            pltpu.VMEM((GR, H), jnp.float32),
            pltpu.VMEM((GR, H), jnp.float32),
            pltpu.VMEM((GR, H), jnp.float32),
            pltpu.VMEM((GR, H), jnp.float32),
            pltpu.SemaphoreType.DMA,
            pltpu.SemaphoreType.DMA,
            pltpu.SemaphoreType.DMA,
            pltpu.SemaphoreType.DMA,
        ],
    )
    def gather(a_hbm, b_hbm, i0_hbm, i1_hbm, g0_hbm, g1_hbm,
               i0_v, i1_v, r0a, r0b, r1a, r1b, gs0, gs1, ws0, ws1):
        c = lax.axis_index("c")
        s = lax.axis_index("s")
        wid = c * NS + s
        pltpu.sync_copy(i0_hbm.at[wid], i0_v)
        pltpu.sync_copy(i1_hbm.at[wid], i1_v)
        bufs0 = (r0a, r0b)
        bufs1 = (r1a, r1b)

        def issue(t, b):
            d0 = pltpu.async_copy(a_hbm.at[i0_v.at[pl.ds(t * GR, GR)]],
                                  bufs0[b], gs0)
            d1 = pltpu.async_copy(b_hbm.at[i1_v.at[pl.ds(t * GR, GR)]],
                                  bufs1[b], gs1)
            return d0, d1

        gd = {0: issue(0, 0)}
        wd = {}
        for t in range(NG):
            b = t % 2
            if t + 1 < NG:
                if t >= 1:
                    wd[t - 1][0].wait()
                    wd[t - 1][1].wait()
                gd[t + 1] = issue(t + 1, 1 - b)
            gd[t][0].wait()
            gd[t][1].wait()
            base = wid * EPT + t * GR
            w0 = pltpu.async_copy(bufs0[b], g0_hbm.at[pl.ds(base, GR)], ws0)
            w1 = pltpu.async_copy(bufs1[b], g1_hbm.at[pl.ds(base, GR)], ws1)
            wd[t] = (w0, w1)
        for t in (NG - 2, NG - 1):
            wd[t][0].wait()
            wd[t][1].wait()

    return gather


@functools.cache
def _sc_scatter_kernel():
    mesh = plsc.VectorSubcoreMesh(core_axis_name="c", subcore_axis_name="s",
                                  num_cores=NC, num_subcores=NS)

    @functools.partial(
        pl.kernel,
        out_type=jax.ShapeDtypeStruct((NC, NSH, MP), jnp.float32),
        mesh=mesh,
        compiler_params=pltpu.CompilerParams(use_tc_tiling_on_sc=False),
        scratch_types=[
            pltpu.VMEM((KCH, CH), jnp.int32),
            pltpu.VMEM((GR, MP), jnp.float32),
            pltpu.VMEM((GR, MP), jnp.float32),
            pltpu.VMEM((RPT, MP), jnp.float32),
            pltpu.VMEM_SHARED((NSH, MP), jnp.float32),
            pltpu.SemaphoreType.DMA,
            pltpu.SemaphoreType.DMA,
        ],
    )
    def scatter(m_hbm, i0_hbm, p_hbm, i0_v, ra, rb, st_v, acc_sh, ls, ss):
        c = lax.axis_index("c")
        s = lax.axis_index("s")
        wid = c * NS + s

        def zrow(i, carry):
            st_v[i, :] = jnp.zeros((MP,), jnp.float32)
            return carry

        lax.fori_loop(0, RPT, zrow, 0)
        pltpu.sync_copy(i0_hbm.at[wid], i0_v)
        pltpu.sync_copy(st_v, acc_sh.at[pl.ds(s * RPT, RPT)])
        plsc.subcore_barrier()

        bufs = (ra, rb)

        def load(t, b):
            return pltpu.async_copy(
                m_hbm.at[pl.ds(wid * EPT + t * GR, GR)], bufs[b], ls)

        ld = {0: load(0, 0)}
        sd = {}
        for t in range(NG):
            b = t % 2
            if t + 1 < NG:
                if t >= 1:
                    for k in range(GK):
                        sd[(t - 1, k)].wait()
                ld[t + 1] = load(t + 1, 1 - b)
            ld[t].wait()
            for k in range(GK):
                sd[(t, k)] = pltpu.async_copy(
                    bufs[b].at[pl.ds(k * CH, CH)],
                    acc_sh.at[i0_v.at[t * GK + k]], ss, add=True)
        for t in (NG - 2, NG - 1):
            for k in range(GK):
                sd[(t, k)].wait()
        plsc.subcore_barrier()

        pltpu.sync_copy(acc_sh.at[pl.ds(s * RPT, RPT)], st_v)
        pltpu.sync_copy(st_v, p_hbm.at[c, pl.ds(s * RPT, RPT)])

    return scatter


def _sc_gather(a, b, i0, i1):
    return _sc_gather_kernel()(a, b, i0, i1)


def _sc_scatter(m, i0s):
    return _sc_scatter_kernel()(m, i0s)


_BN = 2000
_BE = 2048


def _proj_body(x_ref, wa_ref, wb_ref, a_ref, b_ref):
    xb = x_ref[...]
    a_ref[...] = jnp.dot(xb, wa_ref[...], preferred_element_type=jnp.float32)
    b_ref[...] = jnp.dot(xb, wb_ref[...], preferred_element_type=jnp.float32)


def _proj(h, wa, wb):
    return pl.pallas_call(
        _proj_body,
        grid=(N // _BN,),
        in_specs=[
            pl.BlockSpec((_BN, D), lambda i: (i, 0)),
            pl.BlockSpec((D, H), lambda i: (0, 0)),
            pl.BlockSpec((D, H), lambda i: (0, 0)),
        ],
        out_specs=[
            pl.BlockSpec((_BN, H), lambda i: (i, 0)),
            pl.BlockSpec((_BN, H), lambda i: (i, 0)),
        ],
        out_shape=[
            jax.ShapeDtypeStruct((N, H), jnp.float32),
            jax.ShapeDtypeStruct((N, H), jnp.float32),
        ],
    )(h, wa, wb)


def _edge_body(g0_ref, g1_ref, ea_ref, w1c_ref, b1_ref, w2_ref, b2_ref, m_ref):
    sv = (g0_ref[...] + g1_ref[...]
          + jnp.dot(ea_ref[...], w1c_ref[...], preferred_element_type=jnp.float32)
          + b1_ref[...])
    t = _silu(sv)
    m_ref[...] = _silu(jnp.dot(t, w2_ref[...], preferred_element_type=jnp.float32)
                       + b2_ref[...])


def _edge(g0, g1, eap, w1c, b1, w2p, b2p):
    return pl.pallas_call(
        _edge_body,
        grid=(EP // _BE,),
        in_specs=[
            pl.BlockSpec((_BE, H), lambda i: (i, 0)),
            pl.BlockSpec((_BE, H), lambda i: (i, 0)),
            pl.BlockSpec((_BE, DE), lambda i: (i, 0)),
            pl.BlockSpec((DE, H), lambda i: (0, 0)),
            pl.BlockSpec((1, H), lambda i: (0, 0)),
            pl.BlockSpec((H, MP), lambda i: (0, 0)),
            pl.BlockSpec((1, MP), lambda i: (0, 0)),
        ],
        out_specs=pl.BlockSpec((_BE, MP), lambda i: (i, 0)),
        out_shape=jax.ShapeDtypeStruct((EP, MP), jnp.float32),
    )(g0, g1, eap, w1c, b1, w2p, b2p)


def _node_proj_body(h_ref, p0_ref, p1_ref, v1a_ref, v1b_ref, b1_ref,
                    w2_ref, b2_ref, wa_ref, wb_ref, h_out, a_out, b_out):
    hb = h_ref[...]
    sums = p0_ref[0] + p1_ref[0]
    t = _silu(jnp.dot(hb, v1a_ref[...], preferred_element_type=jnp.float32)
              + jnp.dot(sums, v1b_ref[...], preferred_element_type=jnp.float32)
              + b1_ref[...])
    hn = _silu(_silu(jnp.dot(t, w2_ref[...], preferred_element_type=jnp.float32)
                     + b2_ref[...]))
    h_out[...] = hn
    a_out[...] = jnp.dot(hn, wa_ref[...], preferred_element_type=jnp.float32)
    b_out[...] = jnp.dot(hn, wb_ref[...], preferred_element_type=jnp.float32)


def _node_final_body(h_ref, p0_ref, p1_ref, v1a_ref, v1b_ref, b1_ref,
                     w2_ref, b2_ref, h_out):
    hb = h_ref[...]
    sums = p0_ref[0] + p1_ref[0]
    t = _silu(jnp.dot(hb, v1a_ref[...], preferred_element_type=jnp.float32)
              + jnp.dot(sums, v1b_ref[...], preferred_element_type=jnp.float32)
              + b1_ref[...])
    h_out[...] = _silu(_silu(jnp.dot(t, w2_ref[...], preferred_element_type=jnp.float32)
                             + b2_ref[...]))


def _node_specs():
    return [
        pl.BlockSpec((_BN, D), lambda i: (i, 0)),
        pl.BlockSpec((1, _BN, MP), lambda i: (0, i, 0)),
        pl.BlockSpec((1, _BN, MP), lambda i: (1, i, 0)),
        pl.BlockSpec((D, H), lambda i: (0, 0)),
        pl.BlockSpec((MP, H), lambda i: (0, 0)),
        pl.BlockSpec((1, H), lambda i: (0, 0)),
        pl.BlockSpec((H, D), lambda i: (0, 0)),
        pl.BlockSpec((1, D), lambda i: (0, 0)),
    ]


def _node_proj(h, p, v1a, v1b, b1, w2, b2, wa, wb):
    return pl.pallas_call(
        _node_proj_body,
        grid=(N // _BN,),
        in_specs=_node_specs() + [
            pl.BlockSpec((D, H), lambda i: (0, 0)),
            pl.BlockSpec((D, H), lambda i: (0, 0)),
        ],
        out_specs=[
            pl.BlockSpec((_BN, D), lambda i: (i, 0)),
            pl.BlockSpec((_BN, H), lambda i: (i, 0)),
            pl.BlockSpec((_BN, H), lambda i: (i, 0)),
        ],
        out_shape=[
            jax.ShapeDtypeStruct((N, D), jnp.float32),
            jax.ShapeDtypeStruct((N, H), jnp.float32),
            jax.ShapeDtypeStruct((N, H), jnp.float32),
        ],
    )(h, p, p, v1a, v1b, b1, w2, b2, wa, wb)


def _node_final(h, p, v1a, v1b, b1, w2, b2):
    return pl.pallas_call(
        _node_final_body,
        grid=(N // _BN,),
        in_specs=_node_specs(),
        out_specs=pl.BlockSpec((_BN, D), lambda i: (i, 0)),
        out_shape=jax.ShapeDtypeStruct((N, D), jnp.float32),
    )(h, p, p, v1a, v1b, b1, w2, b2)


def _final_body(ui_ref, ue_ref, bt_ref, w1a_ref, w1b_ref, b1_ref,
                w2_ref, b2_ref, o_ref, acc_i, acc_e, cnt):
    k = pl.program_id(0)

    @pl.when(k == 0)
    def _():
        acc_i[...] = jnp.zeros_like(acc_i)
        acc_e[...] = jnp.zeros_like(acc_e)
        cnt[...] = jnp.zeros_like(cnt)

    bt = bt_ref[:, :G]
    seg = lax.broadcasted_iota(jnp.int32, (_BN, G), 1)
    oh = jnp.where(seg == bt, 1.0, 0.0)
    dn = (((0,), (0,)), ((), ()))
    acc_i[...] += lax.dot_general(oh, ui_ref[...], dn,
                                  preferred_element_type=jnp.float32)
    acc_e[...] += lax.dot_general(oh, ue_ref[...], dn,
                                  preferred_element_type=jnp.float32)
    cnt[...] += lax.dot_general(oh, jnp.ones((_BN, D), jnp.float32), dn,
                                preferred_element_type=jnp.float32)

    @pl.when(k == (N // _BN) - 1)
    def _():
        c = jnp.maximum(cnt[...], 1.0)
        mi = acc_i[...] / c
        me = acc_e[...] / c
        t = _silu(jnp.dot(mi, w1a_ref[...], preferred_element_type=jnp.float32)
                  + jnp.dot(me, w1b_ref[...], preferred_element_type=jnp.float32)
                  + b1_ref[...])
        o_ref[...] = _silu(jnp.dot(t, w2_ref[...], preferred_element_type=jnp.float32)
                           + b2_ref[...])


def _final(ui, ue, btf, w1a, w1b, b1, w2, b2):
    return pl.pallas_call(
        _final_body,
        grid=(N // _BN,),
        in_specs=[
            pl.BlockSpec((_BN, D), lambda i: (i, 0)),
            pl.BlockSpec((_BN, D), lambda i: (i, 0)),
            pl.BlockSpec((_BN, D), lambda i: (i, 0)),
            pl.BlockSpec((D, H), lambda i: (0, 0)),
            pl.BlockSpec((D, H), lambda i: (0, 0)),
            pl.BlockSpec((1, H), lambda i: (0, 0)),
            pl.BlockSpec((H, OUT), lambda i: (0, 0)),
            pl.BlockSpec((1, OUT), lambda i: (0, 0)),
        ],
        out_specs=pl.BlockSpec((G, OUT), lambda i: (0, 0)),
        out_shape=jax.ShapeDtypeStruct((G, OUT), jnp.float32),
        scratch_shapes=[
            pltpu.VMEM((G, D), jnp.float32),
            pltpu.VMEM((G, D), jnp.float32),
            pltpu.VMEM((G, D), jnp.float32),
        ],
    )(ui, ue, btf, w1a, w1b, b1, w2, b2)


def _prep_conv(p):
    w1 = p["edge"]["W1"]
    wa = w1[:D]
    wb = w1[D:2 * D]
    w1c = w1[2 * D:]
    b1 = p["edge"]["b1"].reshape(1, H)
    w2p = jnp.zeros((H, MP), jnp.float32).at[:, :MSG].set(p["edge"]["W2"])
    b2p = jnp.zeros((1, MP), jnp.float32).at[0, :MSG].set(p["edge"]["b2"])
    nw1 = p["node"]["W1"]
    v1a = nw1[:D]
    v1b = jnp.zeros((MP, H), jnp.float32).at[:MSG].set(nw1[D:])
    nb1 = p["node"]["b1"].reshape(1, H)
    nw2 = p["node"]["W2"]
    nb2 = p["node"]["b2"].reshape(1, D)
    return wa, wb, w1c, b1, w2p, b2p, v1a, v1b, nb1, nw2, nb2


def _branch(x, eidx, eattr, layers):
    pad = EP - E
    n0 = eidx[0].astype(jnp.int32)
    n1 = eidx[1].astype(jnp.int32)
    n0g = jnp.concatenate([n0, jnp.zeros((pad,), jnp.int32)]).reshape(NW, EPT)
    n1g = jnp.concatenate([n1, jnp.zeros((pad,), jnp.int32)]).reshape(NW, EPT)
    n0s = jnp.concatenate([n0, jnp.full((pad,), N, jnp.int32)]).reshape(NW, KCH, CH)
    eap = jnp.concatenate([eattr, jnp.zeros((pad, DE), jnp.float32)])

    prepped = [_prep_conv(p) for p in layers]
    h = x
    a, b = _proj(h, prepped[0][0], prepped[0][1])
    for li, pr in enumerate(prepped):
        wa, wb, w1c, b1, w2p, b2p, v1a, v1b, nb1, nw2, nb2 = pr
        g0, g1 = _sc_gather(a, b, n0g, n1g)
        m = _edge(g0, g1, eap, w1c, b1, w2p, b2p)
        psum = _sc_scatter(m, n0s)
        if li + 1 < len(prepped):
            h, a, b = _node_proj(h, psum, v1a, v1b, nb1, nw2, nb2,
                                 prepped[li + 1][0], prepped[li + 1][1])
        else:
            h = _node_final(h, psum, v1a, v1b, nb1, nw2, nb2)
    return h


def kernel(x, internal_edge_index, internal_edge_attr, edge_index, edge_attr,
           batch, internal_params, external_params, graph_params):
    upd_int = _branch(x, internal_edge_index, internal_edge_attr, internal_params)
    upd_ext = _branch(x, edge_index, edge_attr, external_params)
    btf = jnp.broadcast_to(batch.astype(jnp.int32)[:, None], (N, D))
    gw1 = graph_params["W1"]
    out = _final(upd_int, upd_ext, btf,
                 gw1[:D], gw1[D:], graph_params["b1"].reshape(1, H),
                 graph_params["W2"], graph_params["b2"].reshape(1, OUT))
    return out


# speedup vs baseline: 4.4971x; 1.7658x over previous
import functools
import jax
import jax.numpy as jnp
from jax import lax
from jax.experimental import pallas as pl
from jax.experimental.pallas import tpu as pltpu
from jax.experimental.pallas import tpu_sc as plsc

N = 10000
E = 320000
D = 128
DE = 16
G = 64
H = 32
MSG = 4
OUT = 8

MW = 32
NC = 2
NS = 16
NW = NC * NS
CH = 128
KCH = 80
EPT = KCH * CH
EP = NW * EPT
N4 = N // 4
NSH = N + 48
RPT = NSH // NS

GK = 4
GR = GK * CH
NG = KCH // GK


def _silu(v):
    return v * jax.nn.sigmoid(v)


@functools.cache
def _sc_gather_kernel():
    mesh = plsc.VectorSubcoreMesh(core_axis_name="c", subcore_axis_name="s",
                                  num_cores=NC, num_subcores=NS)

    @functools.partial(
        pl.kernel,
        out_type=(
            jax.ShapeDtypeStruct((EP, H), jnp.float32),
            jax.ShapeDtypeStruct((EP, H), jnp.float32),
        ),
        mesh=mesh,
        compiler_params=pltpu.CompilerParams(use_tc_tiling_on_sc=False),
        scratch_types=[
            pltpu.VMEM((EPT,), jnp.int32),
            pltpu.VMEM((EPT,), jnp.int32),
            pltpu.VMEM((GR, H), jnp.float32),
            pltpu.VMEM((GR, H), jnp.float32),
            pltpu.VMEM((GR, H), jnp.float32),
            pltpu.VMEM((GR, H), jnp.float32),
            pltpu.SemaphoreType.DMA,
            pltpu.SemaphoreType.DMA,
            pltpu.SemaphoreType.DMA,
            pltpu.SemaphoreType.DMA,
        ],
    )
    def gather(a_hbm, b_hbm, i0_hbm, i1_hbm, g0_hbm, g1_hbm,
               i0_v, i1_v, r0a, r0b, r1a, r1b, gs0, gs1, ws0, ws1):
        c = lax.axis_index("c")
        s = lax.axis_index("s")
        wid = c * NS + s
        pltpu.sync_copy(i0_hbm.at[wid], i0_v)
        pltpu.sync_copy(i1_hbm.at[wid], i1_v)
        bufs0 = (r0a, r0b)
        bufs1 = (r1a, r1b)

        def issue(t, b):
            d0 = pltpu.async_copy(a_hbm.at[i0_v.at[pl.ds(t * GR, GR)]],
                                  bufs0[b], gs0)
            d1 = pltpu.async_copy(b_hbm.at[i1_v.at[pl.ds(t * GR, GR)]],
                                  bufs1[b], gs1)
            return d0, d1

        gd = {0: issue(0, 0)}
        wd = {}
        for t in range(NG):
            b = t % 2
            if t + 1 < NG:
                if t >= 1:
                    wd[t - 1][0].wait()
                    wd[t - 1][1].wait()
                gd[t + 1] = issue(t + 1, 1 - b)
            gd[t][0].wait()
            gd[t][1].wait()
            base = wid * EPT + t * GR
            w0 = pltpu.async_copy(bufs0[b], g0_hbm.at[pl.ds(base, GR)], ws0)
            w1 = pltpu.async_copy(bufs1[b], g1_hbm.at[pl.ds(base, GR)], ws1)
            wd[t] = (w0, w1)
        for t in (NG - 2, NG - 1):
            wd[t][0].wait()
            wd[t][1].wait()

    return gather


@functools.cache
def _sc_scatter_kernel():
    mesh = plsc.VectorSubcoreMesh(core_axis_name="c", subcore_axis_name="s",
                                  num_cores=NC, num_subcores=NS)

    @functools.partial(
        pl.kernel,
        out_type=jax.ShapeDtypeStruct((NC, NSH, MW), jnp.float32),
        mesh=mesh,
        compiler_params=pltpu.CompilerParams(use_tc_tiling_on_sc=False),
        scratch_types=[
            pltpu.VMEM((KCH, CH), jnp.int32),
            pltpu.VMEM((GR, MW), jnp.float32),
            pltpu.VMEM((GR, MW), jnp.float32),
            pltpu.VMEM((RPT, MW), jnp.float32),
            pltpu.VMEM_SHARED((NSH, MW), jnp.float32),
            pltpu.SemaphoreType.DMA,
            pltpu.SemaphoreType.DMA,
        ],
    )
    def scatter(m_hbm, i0_hbm, p_hbm, i0_v, ra, rb, st_v, acc_sh, ls, ss):
        c = lax.axis_index("c")
        s = lax.axis_index("s")
        wid = c * NS + s

        def zrow(i, carry):
            st_v[i, pl.ds(0, 16)] = jnp.zeros((16,), jnp.float32)
            st_v[i, pl.ds(16, 16)] = jnp.zeros((16,), jnp.float32)
            return carry

        lax.fori_loop(0, RPT, zrow, 0)
        pltpu.sync_copy(i0_hbm.at[wid], i0_v)
        pltpu.sync_copy(st_v, acc_sh.at[pl.ds(s * RPT, RPT)])
        plsc.subcore_barrier()

        bufs = (ra, rb)

        def load(t, b):
            return pltpu.async_copy(
                m_hbm.at[pl.ds(wid * EPT + t * GR, GR)], bufs[b], ls)

        ld = {0: load(0, 0)}
        sd = {}
        for t in range(NG):
            b = t % 2
            if t + 1 < NG:
                if t >= 1:
                    for k in range(GK):
                        sd[(t - 1, k)].wait()
                ld[t + 1] = load(t + 1, 1 - b)
            ld[t].wait()
            for k in range(GK):
                sd[(t, k)] = pltpu.async_copy(
                    bufs[b].at[pl.ds(k * CH, CH)],
                    acc_sh.at[i0_v.at[t * GK + k]], ss, add=True)
        for t in (NG - 2, NG - 1):
            for k in range(GK):
                sd[(t, k)].wait()
        plsc.subcore_barrier()

        pltpu.sync_copy(acc_sh.at[pl.ds(s * RPT, RPT)], st_v)
        pltpu.sync_copy(st_v, p_hbm.at[c, pl.ds(s * RPT, RPT)])

    return scatter


def _sc_gather(a, b, i0, i1):
    return _sc_gather_kernel()(a, b, i0, i1)


def _sc_scatter(m, i0s):
    return _sc_scatter_kernel()(m, i0s)


EP8 = EP // 8
E8 = E // 8
NSH4 = NSH // 4
_BQ = 400


def _proj_body(h4_ref, wa_ref, wb_ref, a_ref, b_ref):
    h4 = h4_ref[...]
    a_ref[...] = jnp.dot(h4, wa_ref[...], preferred_element_type=jnp.float32)
    b_ref[...] = jnp.dot(h4, wb_ref[...], preferred_element_type=jnp.float32)


def _proj(h4, w4a, w4b):
    return pl.pallas_call(
        _proj_body,
        out_shape=[
            jax.ShapeDtypeStruct((N4, D), jnp.float32),
            jax.ShapeDtypeStruct((N4, D), jnp.float32),
        ],
    )(h4, w4a, w4b)


def _edge_body(g0_ref, g1_ref, ea_ref, w1c_ref, b1_ref, w2_ref, b2_ref, m_ref):
    cc = jnp.dot(ea_ref[...], w1c_ref[...], preferred_element_type=jnp.float32)
    sv = g0_ref[...] + g1_ref[...] + cc + b1_ref[...]
    t = _silu(sv)
    m_ref[...] = _silu(jnp.dot(t, w2_ref[...], preferred_element_type=jnp.float32)
                       + b2_ref[...])


def _edge(g0, g1, ea8, w1c8, b1t8, w2bd8, b2t8):
    return pl.pallas_call(
        _edge_body,
        grid=(E8 // _BQ,),
        in_specs=[
            pl.BlockSpec((_BQ, 2 * D), lambda i: (i, 0)),
            pl.BlockSpec((_BQ, 2 * D), lambda i: (i, 0)),
            pl.BlockSpec((_BQ, D), lambda i: (i, 0)),
            pl.BlockSpec((D, 2 * D), lambda i: (0, 0)),
            pl.BlockSpec((1, 2 * D), lambda i: (0, 0)),
            pl.BlockSpec((2 * D, 2 * D), lambda i: (0, 0)),
            pl.BlockSpec((1, 2 * D), lambda i: (0, 0)),
        ],
        out_specs=pl.BlockSpec((_BQ, 2 * D), lambda i: (i, 0)),
        out_shape=jax.ShapeDtypeStruct((EP8, 2 * D), jnp.float32),
    )(g0, g1, ea8, w1c8, b1t8, w2bd8, b2t8)


def _node_body(h4_ref, p_ref, v4a_ref, v4b_ref, b1_ref, w24_ref, b2_ref,
               h_out):
    h4 = h4_ref[...]
    sums4 = p_ref[0, :N4] + p_ref[1, :N4]
    t = _silu(jnp.dot(h4, v4a_ref[...], preferred_element_type=jnp.float32)
              + jnp.dot(sums4, v4b_ref[...], preferred_element_type=jnp.float32)
              + b1_ref[...])
    h_out[...] = _silu(_silu(jnp.dot(t, w24_ref[...],
                                     preferred_element_type=jnp.float32)
                             + b2_ref[...]))


def _node(h4, p4, v4a, v4b, nb1t4, w24, nb2t4):
    return pl.pallas_call(
        _node_body,
        out_shape=jax.ShapeDtypeStruct((N4, 4 * D), jnp.float32),
    )(h4, p4, v4a, v4b, nb1t4, w24, nb2t4)


def _final_body(ui_ref, ue_ref, bt_ref, w1a_ref, w1b_ref, b1_ref,
                w2_ref, b2_ref, o_ref):
    bt = bt_ref[:, :G]
    seg = lax.broadcasted_iota(jnp.int32, (N, G), 1)
    oh = jnp.where(seg == bt, 1.0, 0.0)
    dn = (((0,), (0,)), ((), ()))
    acc_i = lax.dot_general(oh, ui_ref[...], dn,
                            preferred_element_type=jnp.float32)
    acc_e = lax.dot_general(oh, ue_ref[...], dn,
                            preferred_element_type=jnp.float32)
    cnt = lax.dot_general(oh, jnp.ones((N, D), jnp.float32), dn,
                          preferred_element_type=jnp.float32)
    c = jnp.maximum(cnt, 1.0)
    mi = acc_i / c
    me = acc_e / c
    t = _silu(jnp.dot(mi, w1a_ref[...], preferred_element_type=jnp.float32)
              + jnp.dot(me, w1b_ref[...], preferred_element_type=jnp.float32)
              + b1_ref[...])
    o_ref[...] = _silu(jnp.dot(t, w2_ref[...], preferred_element_type=jnp.float32)
                       + b2_ref[...])


def _final(ui, ue, btf, w1a, w1b, b1, w2, b2):
    return pl.pallas_call(
        _final_body,
        out_shape=jax.ShapeDtypeStruct((G, OUT), jnp.float32),
    )(ui, ue, btf, w1a, w1b, b1, w2, b2)


def _kron4(w):
    return jnp.kron(jnp.eye(4, dtype=jnp.float32), w)


def _kron8(w):
    return jnp.kron(jnp.eye(8, dtype=jnp.float32), w)


def _prep_conv(p):
    w1 = p["edge"]["W1"]
    w4a = _kron4(w1[:D])
    w4b = _kron4(w1[D:2 * D])
    w1c8 = _kron8(w1[2 * D:])
    b1t8 = jnp.tile(p["edge"]["b1"].reshape(1, H), (1, 8))
    w2p = jnp.zeros((H, MW), jnp.float32).at[:, :MSG].set(p["edge"]["W2"])
    w2bd8 = _kron8(w2p)
    b2p = jnp.zeros((1, MW), jnp.float32).at[0, :MSG].set(p["edge"]["b2"])
    b2t8 = jnp.tile(b2p, (1, 8))
    nw1 = p["node"]["W1"]
    v4a = _kron4(nw1[:D])
    v1b = jnp.zeros((MW, H), jnp.float32).at[:MSG].set(nw1[D:])
    v4b = _kron4(v1b)
    nb1t4 = jnp.tile(p["node"]["b1"].reshape(1, H), (1, 4))
    w24 = _kron4(p["node"]["W2"])
    nb2t4 = jnp.tile(p["node"]["b2"].reshape(1, D), (1, 4))
    return w4a, w4b, w1c8, b1t8, w2bd8, b2t8, v4a, v4b, nb1t4, w24, nb2t4


def _branch(x, eidx, eattr, layers):
    pad = EP - E
    n0 = eidx[0].astype(jnp.int32)
    n1 = eidx[1].astype(jnp.int32)
    n0g = jnp.concatenate([n0, jnp.zeros((pad,), jnp.int32)]).reshape(NW, EPT)
    n1g = jnp.concatenate([n1, jnp.zeros((pad,), jnp.int32)]).reshape(NW, EPT)
    n0s = jnp.concatenate([n0, jnp.full((pad,), N, jnp.int32)]).reshape(NW, KCH, CH)
    ea8 = eattr.reshape(E8, D)

    prepped = [_prep_conv(p) for p in layers]
    h4 = x.reshape(N4, 4 * D)
    for pr in prepped:
        w4a, w4b, w1c8, b1t8, w2bd8, b2t8, v4a, v4b, nb1t4, w24, nb2t4 = pr
        a, b = _proj(h4, w4a, w4b)
        g0, g1 = _sc_gather(a.reshape(N, H), b.reshape(N, H), n0g, n1g)
        m8 = _edge(g0.reshape(EP8, 2 * D), g1.reshape(EP8, 2 * D),
                   ea8, w1c8, b1t8, w2bd8, b2t8)
        psum = _sc_scatter(m8.reshape(EP, MW), n0s)
        h4 = _node(h4, psum.reshape(NC, NSH4, D), v4a, v4b, nb1t4, w24, nb2t4)
    return h4.reshape(N, D)


def kernel(x, internal_edge_index, internal_edge_attr, edge_index, edge_attr,
           batch, internal_params, external_params, graph_params):
    upd_int = _branch(x, internal_edge_index, internal_edge_attr, internal_params)
    upd_ext = _branch(x, edge_index, edge_attr, external_params)
    btf = jnp.broadcast_to(batch.astype(jnp.int32)[:, None], (N, D))
    gw1 = graph_params["W1"]
    out = _final(upd_int, upd_ext, btf,
                 gw1[:D], gw1[D:], graph_params["b1"].reshape(1, H),
                 graph_params["W2"], graph_params["b2"].reshape(1, OUT))
    return out


# speedup vs baseline: 4.4972x; 1.0000x over previous
import functools
import jax
import jax.numpy as jnp
from jax import lax
from jax.experimental import pallas as pl
from jax.experimental.pallas import tpu as pltpu
from jax.experimental.pallas import tpu_sc as plsc

N = 10000
E = 320000
D = 128
DE = 16
G = 64
H = 32
MSG = 4
OUT = 8

MW = 32
NC = 2
NS = 16
NW = NC * NS
CH = 128
KCH = 80
EPT = KCH * CH
EP = NW * EPT
N4 = N // 4
NSH = N + 48
RPT = NSH // NS

GK = 4
GR = GK * CH
NG = KCH // GK


def _silu(v):
    return v * jax.nn.sigmoid(v)


@functools.cache
def _sc_gather_kernel():
    mesh = plsc.VectorSubcoreMesh(core_axis_name="c", subcore_axis_name="s",
                                  num_cores=NC, num_subcores=NS)

    @functools.partial(
        pl.kernel,
        out_type=(
            jax.ShapeDtypeStruct((EP, H), jnp.float32),
            jax.ShapeDtypeStruct((EP, H), jnp.float32),
        ),
        mesh=mesh,
        compiler_params=pltpu.CompilerParams(use_tc_tiling_on_sc=False),
        scratch_types=[
            pltpu.VMEM((EPT,), jnp.int32),
            pltpu.VMEM((EPT,), jnp.int32),
            pltpu.VMEM((GR, H), jnp.float32),
            pltpu.VMEM((GR, H), jnp.float32),
            pltpu.VMEM((GR, H), jnp.float32),
            pltpu.VMEM((GR, H), jnp.float32),
            pltpu.SemaphoreType.DMA,
            pltpu.SemaphoreType.DMA,
            pltpu.SemaphoreType.DMA,
            pltpu.SemaphoreType.DMA,
        ],
    )
    def gather(a_hbm, b_hbm, i0_hbm, i1_hbm, g0_hbm, g1_hbm,
               i0_v, i1_v, r0a, r0b, r1a, r1b, gs0, gs1, ws0, ws1):
        c = lax.axis_index("c")
        s = lax.axis_index("s")
        wid = c * NS + s
        pltpu.sync_copy(i0_hbm.at[wid], i0_v)
        pltpu.sync_copy(i1_hbm.at[wid], i1_v)
        bufs0 = (r0a, r0b)
        bufs1 = (r1a, r1b)

        def issue(t, b):
            d0 = pltpu.async_copy(a_hbm.at[i0_v.at[pl.ds(t * GR, GR)]],
                                  bufs0[b], gs0)
            d1 = pltpu.async_copy(b_hbm.at[i1_v.at[pl.ds(t * GR, GR)]],
                                  bufs1[b], gs1)
            return d0, d1

        gd = {0: issue(0, 0)}
        wd = {}
        for t in range(NG):
            b = t % 2
            if t + 1 < NG:
                if t >= 1:
                    wd[t - 1][0].wait()
                    wd[t - 1][1].wait()
                gd[t + 1] = issue(t + 1, 1 - b)
            gd[t][0].wait()
            gd[t][1].wait()
            base = wid * EPT + t * GR
            w0 = pltpu.async_copy(bufs0[b], g0_hbm.at[pl.ds(base, GR)], ws0)
            w1 = pltpu.async_copy(bufs1[b], g1_hbm.at[pl.ds(base, GR)], ws1)
            wd[t] = (w0, w1)
        for t in (NG - 2, NG - 1):
            wd[t][0].wait()
            wd[t][1].wait()

    return gather


@functools.cache
def _sc_scatter_kernel():
    mesh = plsc.VectorSubcoreMesh(core_axis_name="c", subcore_axis_name="s",
                                  num_cores=NC, num_subcores=NS)

    @functools.partial(
        pl.kernel,
        out_type=jax.ShapeDtypeStruct((NC, NSH, MW), jnp.float32),
        mesh=mesh,
        compiler_params=pltpu.CompilerParams(use_tc_tiling_on_sc=False),
        scratch_types=[
            pltpu.VMEM((KCH, CH), jnp.int32),
            pltpu.VMEM((GR, MW), jnp.float32),
            pltpu.VMEM((GR, MW), jnp.float32),
            pltpu.VMEM((RPT, MW), jnp.float32),
            pltpu.VMEM_SHARED((NSH, MW), jnp.float32),
            pltpu.SemaphoreType.DMA,
            pltpu.SemaphoreType.DMA,
        ],
    )
    def scatter(m_hbm, i0_hbm, p_hbm, i0_v, ra, rb, st_v, acc_sh, ls, ss):
        c = lax.axis_index("c")
        s = lax.axis_index("s")
        wid = c * NS + s

        def zrow(i, carry):
            st_v[i, pl.ds(0, 16)] = jnp.zeros((16,), jnp.float32)
            st_v[i, pl.ds(16, 16)] = jnp.zeros((16,), jnp.float32)
            return carry

        lax.fori_loop(0, RPT, zrow, 0)
        pltpu.sync_copy(i0_hbm.at[wid], i0_v)
        pltpu.sync_copy(st_v, acc_sh.at[pl.ds(s * RPT, RPT)])
        plsc.subcore_barrier()

        bufs = (ra, rb)

        def load(t, b):
            return pltpu.async_copy(
                m_hbm.at[pl.ds(wid * EPT + t * GR, GR)], bufs[b], ls)

        ld = {0: load(0, 0)}
        sd = {}
        for t in range(NG):
            b = t % 2
            if t + 1 < NG:
                if t >= 1:
                    for k in range(GK):
                        sd[(t - 1, k)].wait()
                ld[t + 1] = load(t + 1, 1 - b)
            ld[t].wait()
            for k in range(GK):
                sd[(t, k)] = pltpu.async_copy(
                    bufs[b].at[pl.ds(k * CH, CH)],
                    acc_sh.at[i0_v.at[t * GK + k]], ss, add=True)
        for t in (NG - 2, NG - 1):
            for k in range(GK):
                sd[(t, k)].wait()
        plsc.subcore_barrier()

        pltpu.sync_copy(acc_sh.at[pl.ds(s * RPT, RPT)], st_v)
        pltpu.sync_copy(st_v, p_hbm.at[c, pl.ds(s * RPT, RPT)])

    return scatter


def _sc_gather(a, b, i0, i1):
    return _sc_gather_kernel()(a, b, i0, i1)


def _sc_scatter(m, i0s):
    return _sc_scatter_kernel()(m, i0s)


EP8 = EP // 8
E8 = E // 8
NSH4 = NSH // 4
_BQ = 400


def _proj_body(h4_ref, wa_ref, wb_ref, a_ref, b_ref):
    h4 = h4_ref[...]
    a_ref[...] = jnp.dot(h4, wa_ref[...], preferred_element_type=jnp.float32)
    b_ref[...] = jnp.dot(h4, wb_ref[...], preferred_element_type=jnp.float32)


def _proj(h4, w4a, w4b):
    return pl.pallas_call(
        _proj_body,
        out_shape=[
            jax.ShapeDtypeStruct((N4, D), jnp.float32),
            jax.ShapeDtypeStruct((N4, D), jnp.float32),
        ],
    )(h4, w4a, w4b)


def _edge_body(g0_ref, g1_ref, ea_ref, w1c_ref, b1_ref, w2_ref, b2_ref, m_ref):
    cc = jnp.dot(ea_ref[...], w1c_ref[...], preferred_element_type=jnp.float32)
    sv = g0_ref[...] + g1_ref[...] + cc + b1_ref[...]
    t = _silu(sv)
    m_ref[...] = _silu(jnp.dot(t, w2_ref[...], preferred_element_type=jnp.float32)
                       + b2_ref[...])


def _edge(g0, g1, ea8, w1c8, b1t8, w2bd8, b2t8):
    return pl.pallas_call(
        _edge_body,
        grid=(E8 // _BQ,),
        in_specs=[
            pl.BlockSpec((_BQ, 2 * D), lambda i: (i, 0)),
            pl.BlockSpec((_BQ, 2 * D), lambda i: (i, 0)),
            pl.BlockSpec((_BQ, D), lambda i: (i, 0)),
            pl.BlockSpec((D, 2 * D), lambda i: (0, 0)),
            pl.BlockSpec((1, 2 * D), lambda i: (0, 0)),
            pl.BlockSpec((2 * D, 2 * D), lambda i: (0, 0)),
            pl.BlockSpec((1, 2 * D), lambda i: (0, 0)),
        ],
        out_specs=pl.BlockSpec((_BQ, 2 * D), lambda i: (i, 0)),
        out_shape=jax.ShapeDtypeStruct((EP8, 2 * D), jnp.float32),
    )(g0, g1, ea8, w1c8, b1t8, w2bd8, b2t8)


def _node_body(h4_ref, p_ref, v4a_ref, v4b_ref, b1_ref, w24_ref, b2_ref,
               h_out):
    h4 = h4_ref[...]
    sums4 = p_ref[0, :N4] + p_ref[1, :N4]
    t = _silu(jnp.dot(h4, v4a_ref[...], preferred_element_type=jnp.float32)
              + jnp.dot(sums4, v4b_ref[...], preferred_element_type=jnp.float32)
              + b1_ref[...])
    h_out[...] = _silu(_silu(jnp.dot(t, w24_ref[...],
                                     preferred_element_type=jnp.float32)
                             + b2_ref[...]))


def _node(h4, p4, v4a, v4b, nb1t4, w24, nb2t4):
    return pl.pallas_call(
        _node_body,
        out_shape=jax.ShapeDtypeStruct((N4, 4 * D), jnp.float32),
    )(h4, p4, v4a, v4b, nb1t4, w24, nb2t4)


def _final_body(ui_ref, ue_ref, bt_ref, w1a_ref, w1b_ref, b1_ref,
                w2_ref, b2_ref, o_ref):
    bt = bt_ref[:, :G]
    seg = lax.broadcasted_iota(jnp.int32, (N, G), 1)
    oh = jnp.where(seg == bt, 1.0, 0.0)
    dn = (((0,), (0,)), ((), ()))
    acc_i = lax.dot_general(oh, ui_ref[...], dn,
                            preferred_element_type=jnp.float32)
    acc_e = lax.dot_general(oh, ue_ref[...], dn,
                            preferred_element_type=jnp.float32)
    cnt = lax.dot_general(oh, jnp.ones((N, D), jnp.float32), dn,
                          preferred_element_type=jnp.float32)
    c = jnp.maximum(cnt, 1.0)
    mi = acc_i / c
    me = acc_e / c
    t = _silu(jnp.dot(mi, w1a_ref[...], preferred_element_type=jnp.float32)
              + jnp.dot(me, w1b_ref[...], preferred_element_type=jnp.float32)
              + b1_ref[...])
    o_ref[...] = _silu(jnp.dot(t, w2_ref[...], preferred_element_type=jnp.float32)
                       + b2_ref[...])


def _final(ui, ue, btf, w1a, w1b, b1, w2, b2):
    return pl.pallas_call(
        _final_body,
        out_shape=jax.ShapeDtypeStruct((G, OUT), jnp.float32),
    )(ui, ue, btf, w1a, w1b, b1, w2, b2)


def _kron4(w):
    return jnp.kron(jnp.eye(4, dtype=jnp.float32), w)


def _kron8(w):
    return jnp.kron(jnp.eye(8, dtype=jnp.float32), w)


def _prep_conv(p):
    w1 = p["edge"]["W1"]
    w4a = _kron4(w1[:D])
    w4b = _kron4(w1[D:2 * D])
    w1c8 = _kron8(w1[2 * D:])
    b1t8 = jnp.tile(p["edge"]["b1"].reshape(1, H), (1, 8))
    w2p = jnp.zeros((H, MW), jnp.float32).at[:, :MSG].set(p["edge"]["W2"])
    w2bd8 = _kron8(w2p)
    b2p = jnp.zeros((1, MW), jnp.float32).at[0, :MSG].set(p["edge"]["b2"])
    b2t8 = jnp.tile(b2p, (1, 8))
    nw1 = p["node"]["W1"]
    v4a = _kron4(nw1[:D])
    v1b = jnp.zeros((MW, H), jnp.float32).at[:MSG].set(nw1[D:])
    v4b = _kron4(v1b)
    nb1t4 = jnp.tile(p["node"]["b1"].reshape(1, H), (1, 4))
    w24 = _kron4(p["node"]["W2"])
    nb2t4 = jnp.tile(p["node"]["b2"].reshape(1, D), (1, 4))
    return w4a, w4b, w1c8, b1t8, w2bd8, b2t8, v4a, v4b, nb1t4, w24, nb2t4


def _edge_setup(eidx, eattr):
    pad = EP - E
    n0 = eidx[0].astype(jnp.int32)
    n1 = eidx[1].astype(jnp.int32)
    n0g = jnp.concatenate([n0, jnp.zeros((pad,), jnp.int32)]).reshape(NW, EPT)
    n1g = jnp.concatenate([n1, jnp.zeros((pad,), jnp.int32)]).reshape(NW, EPT)
    n0s = jnp.concatenate([n0, jnp.full((pad,), N, jnp.int32)]).reshape(NW, KCH, CH)
    ea8 = eattr.reshape(E8, D)
    return n0g, n1g, n0s, ea8


def _layer(h4, setup, pr):
    n0g, n1g, n0s, ea8 = setup
    w4a, w4b, w1c8, b1t8, w2bd8, b2t8, v4a, v4b, nb1t4, w24, nb2t4 = pr
    a, b = _proj(h4, w4a, w4b)
    g0, g1 = _sc_gather(a.reshape(N, H), b.reshape(N, H), n0g, n1g)
    m8 = _edge(g0.reshape(EP8, 2 * D), g1.reshape(EP8, 2 * D),
               ea8, w1c8, b1t8, w2bd8, b2t8)
    psum = _sc_scatter(m8.reshape(EP, MW), n0s)
    return _node(h4, psum.reshape(NC, NSH4, D), v4a, v4b, nb1t4, w24, nb2t4)


def kernel(x, internal_edge_index, internal_edge_attr, edge_index, edge_attr,
           batch, internal_params, external_params, graph_params):
    setup_i = _edge_setup(internal_edge_index, internal_edge_attr)
    setup_e = _edge_setup(edge_index, edge_attr)
    prep_i = [_prep_conv(p) for p in internal_params]
    prep_e = [_prep_conv(p) for p in external_params]
    h4i = x.reshape(N4, 4 * D)
    h4e = h4i
    for li in range(len(prep_i)):
        h4i = _layer(h4i, setup_i, prep_i[li])
        h4e = _layer(h4e, setup_e, prep_e[li])
    upd_int = h4i.reshape(N, D)
    upd_ext = h4e.reshape(N, D)
    btf = jnp.broadcast_to(batch.astype(jnp.int32)[:, None], (N, D))
    gw1 = graph_params["W1"]
    out = _final(upd_int, upd_ext, btf,
                 gw1[:D], gw1[D:], graph_params["b1"].reshape(1, H),
                 graph_params["W2"], graph_params["b2"].reshape(1, OUT))
    return out


# speedup vs baseline: 4.9772x; 1.1067x over previous
import functools
import jax
import jax.numpy as jnp
from jax import lax
from jax.experimental import pallas as pl
from jax.experimental.pallas import tpu as pltpu
from jax.experimental.pallas import tpu_sc as plsc

N = 10000
E = 320000
D = 128
DE = 16
G = 64
H = 32
MSG = 4
OUT = 8

MW = 32
NC = 2
NS = 16
NW = NC * NS
CH = 128
KCH = 80
EPT = KCH * CH
EP = NW * EPT
N4 = N // 4
NSH = N + 48
RPT = NSH // NS

GK = 4
GR = GK * CH
NG = KCH // GK


def _silu(v):
    return v * jax.nn.sigmoid(v)


@functools.cache
def _sc_gather_kernel():
    mesh = plsc.VectorSubcoreMesh(core_axis_name="c", subcore_axis_name="s",
                                  num_cores=NC, num_subcores=NS)

    @functools.partial(
        pl.kernel,
        out_type=(
            jax.ShapeDtypeStruct((EP, H), jnp.bfloat16),
            jax.ShapeDtypeStruct((EP, H), jnp.bfloat16),
        ),
        mesh=mesh,
        compiler_params=pltpu.CompilerParams(use_tc_tiling_on_sc=False),
        scratch_types=[
            pltpu.VMEM((EPT,), jnp.int32),
            pltpu.VMEM((EPT,), jnp.int32),
            pltpu.VMEM((GR, H), jnp.bfloat16),
            pltpu.VMEM((GR, H), jnp.bfloat16),
            pltpu.VMEM((GR, H), jnp.bfloat16),
            pltpu.VMEM((GR, H), jnp.bfloat16),
            pltpu.SemaphoreType.DMA,
            pltpu.SemaphoreType.DMA,
            pltpu.SemaphoreType.DMA,
            pltpu.SemaphoreType.DMA,
        ],
    )
    def gather(a_hbm, b_hbm, i0_hbm, i1_hbm, g0_hbm, g1_hbm,
               i0_v, i1_v, r0a, r0b, r1a, r1b, gs0, gs1, ws0, ws1):
        c = lax.axis_index("c")
        s = lax.axis_index("s")
        wid = c * NS + s
        pltpu.sync_copy(i0_hbm.at[wid], i0_v)
        pltpu.sync_copy(i1_hbm.at[wid], i1_v)
        bufs0 = (r0a, r0b)
        bufs1 = (r1a, r1b)

        def issue(t, b):
            d0 = pltpu.async_copy(a_hbm.at[i0_v.at[pl.ds(t * GR, GR)]],
                                  bufs0[b], gs0)
            d1 = pltpu.async_copy(b_hbm.at[i1_v.at[pl.ds(t * GR, GR)]],
                                  bufs1[b], gs1)
            return d0, d1

        gd = {0: issue(0, 0)}
        wd = {}
        for t in range(NG):
            b = t % 2
            if t + 1 < NG:
                if t >= 1:
                    wd[t - 1][0].wait()
                    wd[t - 1][1].wait()
                gd[t + 1] = issue(t + 1, 1 - b)
            gd[t][0].wait()
            gd[t][1].wait()
            base = wid * EPT + t * GR
            w0 = pltpu.async_copy(bufs0[b], g0_hbm.at[pl.ds(base, GR)], ws0)
            w1 = pltpu.async_copy(bufs1[b], g1_hbm.at[pl.ds(base, GR)], ws1)
            wd[t] = (w0, w1)
        for t in (NG - 2, NG - 1):
            wd[t][0].wait()
            wd[t][1].wait()

    return gather


@functools.cache
def _sc_scatter_kernel():
    mesh = plsc.VectorSubcoreMesh(core_axis_name="c", subcore_axis_name="s",
                                  num_cores=NC, num_subcores=NS)

    @functools.partial(
        pl.kernel,
        out_type=jax.ShapeDtypeStruct((NC, NSH, MW), jnp.float32),
        mesh=mesh,
        compiler_params=pltpu.CompilerParams(use_tc_tiling_on_sc=False),
        scratch_types=[
            pltpu.VMEM((KCH, CH), jnp.int32),
            pltpu.VMEM((GR, MW), jnp.float32),
            pltpu.VMEM((GR, MW), jnp.float32),
            pltpu.VMEM((RPT, MW), jnp.float32),
            pltpu.VMEM_SHARED((NSH, MW), jnp.float32),
            pltpu.SemaphoreType.DMA,
            pltpu.SemaphoreType.DMA,
        ],
    )
    def scatter(m_hbm, i0_hbm, p_hbm, i0_v, ra, rb, st_v, acc_sh, ls, ss):
        c = lax.axis_index("c")
        s = lax.axis_index("s")
        wid = c * NS + s

        def zrow(i, carry):
            st_v[i, pl.ds(0, 16)] = jnp.zeros((16,), jnp.float32)
            st_v[i, pl.ds(16, 16)] = jnp.zeros((16,), jnp.float32)
            return carry

        lax.fori_loop(0, RPT, zrow, 0)
        pltpu.sync_copy(i0_hbm.at[wid], i0_v)
        pltpu.sync_copy(st_v, acc_sh.at[pl.ds(s * RPT, RPT)])
        plsc.subcore_barrier()

        bufs = (ra, rb)

        def load(t, b):
            return pltpu.async_copy(
                m_hbm.at[pl.ds(wid * EPT + t * GR, GR)], bufs[b], ls)

        ld = {0: load(0, 0)}
        sd = {}
        for t in range(NG):
            b = t % 2
            if t + 1 < NG:
                if t >= 1:
                    for k in range(GK):
                        sd[(t - 1, k)].wait()
                ld[t + 1] = load(t + 1, 1 - b)
            ld[t].wait()
            for k in range(GK):
                sd[(t, k)] = pltpu.async_copy(
                    bufs[b].at[pl.ds(k * CH, CH)],
                    acc_sh.at[i0_v.at[t * GK + k]], ss, add=True)
        for t in (NG - 2, NG - 1):
            for k in range(GK):
                sd[(t, k)].wait()
        plsc.subcore_barrier()

        pltpu.sync_copy(acc_sh.at[pl.ds(s * RPT, RPT)], st_v)
        pltpu.sync_copy(st_v, p_hbm.at[c, pl.ds(s * RPT, RPT)])

    return scatter


def _sc_gather(a, b, i0, i1):
    return _sc_gather_kernel()(a, b, i0, i1)


def _sc_scatter(m, i0s):
    return _sc_scatter_kernel()(m, i0s)


EP8 = EP // 8
E8 = E // 8
NSH4 = NSH // 4
_BQ = 400


def _proj_body(h4_ref, wa_ref, wb_ref, a_ref, b_ref):
    h4 = h4_ref[...]
    a_ref[...] = jnp.dot(h4, wa_ref[...],
                         preferred_element_type=jnp.float32).astype(jnp.bfloat16)
    b_ref[...] = jnp.dot(h4, wb_ref[...],
                         preferred_element_type=jnp.float32).astype(jnp.bfloat16)


def _proj(h4, w4a, w4b):
    return pl.pallas_call(
        _proj_body,
        out_shape=[
            jax.ShapeDtypeStruct((N4, D), jnp.bfloat16),
            jax.ShapeDtypeStruct((N4, D), jnp.bfloat16),
        ],
    )(h4, w4a, w4b)


def _edge_body(g0_ref, g1_ref, ea_ref, w1c_ref, b1_ref, w2_ref, b2_ref, m_ref):
    cc = jnp.dot(ea_ref[...], w1c_ref[...], preferred_element_type=jnp.float32)
    sv = (g0_ref[...].astype(jnp.float32) + g1_ref[...].astype(jnp.float32)
          + cc + b1_ref[...])
    t = _silu(sv)
    m_ref[...] = _silu(jnp.dot(t, w2_ref[...], preferred_element_type=jnp.float32)
                       + b2_ref[...])


def _edge(g0, g1, ea8, w1c8, b1t8, w2bd8, b2t8):
    return pl.pallas_call(
        _edge_body,
        grid=(E8 // _BQ,),
        in_specs=[
            pl.BlockSpec((_BQ, 2 * D), lambda i: (i, 0)),
            pl.BlockSpec((_BQ, 2 * D), lambda i: (i, 0)),
            pl.BlockSpec((_BQ, D), lambda i: (i, 0)),
            pl.BlockSpec((D, 2 * D), lambda i: (0, 0)),
            pl.BlockSpec((1, 2 * D), lambda i: (0, 0)),
            pl.BlockSpec((2 * D, 2 * D), lambda i: (0, 0)),
            pl.BlockSpec((1, 2 * D), lambda i: (0, 0)),
        ],
        out_specs=pl.BlockSpec((_BQ, 2 * D), lambda i: (i, 0)),
        out_shape=jax.ShapeDtypeStruct((EP8, 2 * D), jnp.float32),
    )(g0, g1, ea8, w1c8, b1t8, w2bd8, b2t8)


def _node_body(h4_ref, p_ref, v4a_ref, v4b_ref, b1_ref, w24_ref, b2_ref,
               h_out):
    h4 = h4_ref[...]
    sums4 = p_ref[0, :N4] + p_ref[1, :N4]
    t = _silu(jnp.dot(h4, v4a_ref[...], preferred_element_type=jnp.float32)
              + jnp.dot(sums4, v4b_ref[...], preferred_element_type=jnp.float32)
              + b1_ref[...])
    h_out[...] = _silu(_silu(jnp.dot(t, w24_ref[...],
                                     preferred_element_type=jnp.float32)
                             + b2_ref[...]))


def _node(h4, p4, v4a, v4b, nb1t4, w24, nb2t4):
    return pl.pallas_call(
        _node_body,
        out_shape=jax.ShapeDtypeStruct((N4, 4 * D), jnp.float32),
    )(h4, p4, v4a, v4b, nb1t4, w24, nb2t4)


def _final_body(ui_ref, ue_ref, bt_ref, w1a_ref, w1b_ref, b1_ref,
                w2_ref, b2_ref, o_ref):
    bt = bt_ref[:, :G]
    seg = lax.broadcasted_iota(jnp.int32, (N, G), 1)
    oh = jnp.where(seg == bt, 1.0, 0.0)
    dn = (((0,), (0,)), ((), ()))
    acc_i = lax.dot_general(oh, ui_ref[...], dn,
                            preferred_element_type=jnp.float32)
    acc_e = lax.dot_general(oh, ue_ref[...], dn,
                            preferred_element_type=jnp.float32)
    cnt = lax.dot_general(oh, jnp.ones((N, D), jnp.float32), dn,
                          preferred_element_type=jnp.float32)
    c = jnp.maximum(cnt, 1.0)
    mi = acc_i / c
    me = acc_e / c
    t = _silu(jnp.dot(mi, w1a_ref[...], preferred_element_type=jnp.float32)
              + jnp.dot(me, w1b_ref[...], preferred_element_type=jnp.float32)
              + b1_ref[...])
    o_ref[...] = _silu(jnp.dot(t, w2_ref[...], preferred_element_type=jnp.float32)
                       + b2_ref[...])


def _final(ui, ue, btf, w1a, w1b, b1, w2, b2):
    return pl.pallas_call(
        _final_body,
        out_shape=jax.ShapeDtypeStruct((G, OUT), jnp.float32),
    )(ui, ue, btf, w1a, w1b, b1, w2, b2)


def _kron4(w):
    return jnp.kron(jnp.eye(4, dtype=jnp.float32), w)


def _kron8(w):
    return jnp.kron(jnp.eye(8, dtype=jnp.float32), w)


def _prep_conv(p):
    w1 = p["edge"]["W1"]
    w4a = _kron4(w1[:D])
    w4b = _kron4(w1[D:2 * D])
    w1c8 = _kron8(w1[2 * D:])
    b1t8 = jnp.tile(p["edge"]["b1"].reshape(1, H), (1, 8))
    w2p = jnp.zeros((H, MW), jnp.float32).at[:, :MSG].set(p["edge"]["W2"])
    w2bd8 = _kron8(w2p)
    b2p = jnp.zeros((1, MW), jnp.float32).at[0, :MSG].set(p["edge"]["b2"])
    b2t8 = jnp.tile(b2p, (1, 8))
    nw1 = p["node"]["W1"]
    v4a = _kron4(nw1[:D])
    v1b = jnp.zeros((MW, H), jnp.float32).at[:MSG].set(nw1[D:])
    v4b = _kron4(v1b)
    nb1t4 = jnp.tile(p["node"]["b1"].reshape(1, H), (1, 4))
    w24 = _kron4(p["node"]["W2"])
    nb2t4 = jnp.tile(p["node"]["b2"].reshape(1, D), (1, 4))
    return w4a, w4b, w1c8, b1t8, w2bd8, b2t8, v4a, v4b, nb1t4, w24, nb2t4


def _edge_setup(eidx, eattr):
    pad = EP - E
    n0 = eidx[0].astype(jnp.int32)
    n1 = eidx[1].astype(jnp.int32)
    n0g = jnp.concatenate([n0, jnp.zeros((pad,), jnp.int32)]).reshape(NW, EPT)
    n1g = jnp.concatenate([n1, jnp.zeros((pad,), jnp.int32)]).reshape(NW, EPT)
    n0s = jnp.concatenate([n0, jnp.full((pad,), N, jnp.int32)]).reshape(NW, KCH, CH)
    ea8 = eattr.reshape(E8, D)
    return n0g, n1g, n0s, ea8


def _layer(h4, setup, pr):
    n0g, n1g, n0s, ea8 = setup
    w4a, w4b, w1c8, b1t8, w2bd8, b2t8, v4a, v4b, nb1t4, w24, nb2t4 = pr
    a, b = _proj(h4, w4a, w4b)
    g0, g1 = _sc_gather(a.reshape(N, H), b.reshape(N, H), n0g, n1g)
    m8 = _edge(g0.reshape(EP8, 2 * D), g1.reshape(EP8, 2 * D),
               ea8, w1c8, b1t8, w2bd8, b2t8)
    psum = _sc_scatter(m8.reshape(EP, MW), n0s)
    return _node(h4, psum.reshape(NC, NSH4, D), v4a, v4b, nb1t4, w24, nb2t4)


def kernel(x, internal_edge_index, internal_edge_attr, edge_index, edge_attr,
           batch, internal_params, external_params, graph_params):
    setup_i = _edge_setup(internal_edge_index, internal_edge_attr)
    setup_e = _edge_setup(edge_index, edge_attr)
    prep_i = [_prep_conv(p) for p in internal_params]
    prep_e = [_prep_conv(p) for p in external_params]
    h4i = x.reshape(N4, 4 * D)
    h4e = h4i
    for li in range(len(prep_i)):
        h4i = _layer(h4i, setup_i, prep_i[li])
        h4e = _layer(h4e, setup_e, prep_e[li])
    upd_int = h4i.reshape(N, D)
    upd_ext = h4e.reshape(N, D)
    btf = jnp.broadcast_to(batch.astype(jnp.int32)[:, None], (N, D))
    gw1 = graph_params["W1"]
    out = _final(upd_int, upd_ext, btf,
                 gw1[:D], gw1[D:], graph_params["b1"].reshape(1, H),
                 graph_params["W2"], graph_params["b2"].reshape(1, OUT))
    return out
